# Initial kernel scaffold; baseline (speedup 1.0000x reference)
import jax
import jax.numpy as jnp
from jax.experimental import pallas as pl


def kernel(x, level):
    raise NotImplementedError("write your pallas kernel here")


# speedup vs baseline: 10.7522x; 10.7522x over previous
import jax
import jax.numpy as jnp
from jax import lax
from jax.experimental import pallas as pl
from jax.experimental.pallas import tpu as pltpu

_THRESH = 3000.0
_HIGHEST = jax.lax.Precision.HIGHEST


def _block_mat(n, m):
    r = lax.broadcasted_iota(jnp.int32, (n, m), 0)
    c = lax.broadcasted_iota(jnp.int32, (n, m), 1)
    return (r // (n // m) == c).astype(jnp.float32)


def _block_mat_t(m, n):
    r = lax.broadcasted_iota(jnp.int32, (m, n), 0)
    c = lax.broadcasted_iota(jnp.int32, (m, n), 1)
    return (c // (n // m) == r).astype(jnp.float32)


def _dot(a, b):
    return lax.dot(a, b, precision=_HIGHEST, preferred_element_type=jnp.float32)


def _qt_body(thr_ref, x_ref, o_ref):
    x = x_ref[0]

    p = _block_mat(512, 8)
    pt = _block_mat_t(8, 512)
    u = _block_mat(8, 4)
    ut = _block_mat_t(4, 8)
    e = _block_mat(8, 2)
    et = _block_mat_t(2, 8)
    w = _block_mat(4, 2)
    wt = _block_mat_t(2, 4)

    s64 = _dot(_dot(pt, x), p)
    m64 = s64 * (1.0 / 4096.0)
    mb = _dot(_dot(p, m64), pt)
    d = x - mb
    v64 = _dot(_dot(pt, d * d), p)

    m128 = _dot(_dot(ut, m64), u) * 0.25
    m128e = _dot(_dot(u, m128), ut)
    dm = m64 - m128e
    v128 = _dot(_dot(ut, v64 + 4096.0 * dm * dm), u)

    m256 = _dot(_dot(wt, m128), w) * 0.25
    m256e4 = _dot(_dot(w, m256), wt)
    dm = m128 - m256e4
    v256 = _dot(_dot(wt, v128 + 16384.0 * dm * dm), w)

    m512 = jnp.sum(m256) * 0.25
    dm = m256 - m512
    v512 = jnp.sum(v256) + 65536.0 * jnp.sum(dm * dm)

    s0 = (v512 >= thr_ref[0]).astype(jnp.float32)
    s1 = (v256 >= thr_ref[1]).astype(jnp.float32)
    s2 = (v128 >= thr_ref[2]).astype(jnp.float32)

    s1e = _dot(_dot(e, s1), et)
    s2e = _dot(_dot(u, s2), ut)
    m256e = _dot(_dot(e, m256), et)
    m128e8 = _dot(_dot(u, m128), ut)

    inner = (1.0 - s2e) * m128e8 + s2e * m64
    mid = (1.0 - s1e) * m256e + s1e * inner
    value = (1.0 - s0) * m512 + s0 * mid

    o_ref[0] = _dot(_dot(p, value), pt)


def kernel(x, level):
    b, c, h, w = x.shape
    xr = x.reshape(b, h, w)
    ns = jnp.array([262144.0, 65536.0, 16384.0], dtype=jnp.float32)
    thr = jnp.where(
        jnp.arange(3) == level,
        jnp.float32(jnp.inf),
        (_THRESH * _THRESH) * (ns - 1.0),
    ).astype(jnp.float32)

    out = pl.pallas_call(
        _qt_body,
        grid=(b,),
        in_specs=[
            pl.BlockSpec(memory_space=pltpu.SMEM),
            pl.BlockSpec((1, h, w), lambda i: (i, 0, 0)),
        ],
        out_specs=pl.BlockSpec((1, h, w), lambda i: (i, 0, 0)),
        out_shape=jax.ShapeDtypeStruct((b, h, w), jnp.float32),
    )(thr, xr)
    return out.reshape(b, 1, h, w)
